# Initial kernel scaffold; baseline (speedup 1.0000x reference)
import jax
import jax.numpy as jnp
from jax.experimental import pallas as pl


def kernel(pos, seed, W1, b1, g1, be1, W2, b2, W3, b3, g3, be3, W4, b4, W5, b5, g5, be5, W6, b6):
    raise NotImplementedError("write your pallas kernel here")


# speedup vs baseline: 1.6631x; 1.6631x over previous
import functools

import jax
import jax.numpy as jnp
from jax.experimental import pallas as pl

B = 4
N = 2048
M = 256
K = 8
CENC = 60
L = 10
EPS = 1e-5

NB_A = 256
PB_B = 4096
NB_C = 256
PB_C = NB_C * K

_F32 = jnp.float32


def _posenc(knn, sel, freqs, is_cos):
    xb = jax.lax.dot_general(knn, sel, (((1,), (0,)), ((), ())),
                             preferred_element_type=_F32, precision=jax.lax.Precision.HIGHEST)
    xf = xb * freqs
    return jnp.where(is_cos, jnp.cos(xf), jnp.sin(xf))


def _enc_consts(dtype):
    j = jax.lax.broadcasted_iota(jnp.int32, (1, CENC), 1)
    freqs = jnp.round(jnp.exp(0.6931471805599453 * ((j % 20) % L).astype(dtype)))
    is_cos = (j % 20) >= L
    c = jax.lax.broadcasted_iota(jnp.int32, (3, CENC), 0)
    jc = jax.lax.broadcasted_iota(jnp.int32, (3, CENC), 1)
    sel = (jc // 20 == c).astype(dtype)
    return sel, freqs, is_cos


def _pass_a_kernel(posT_ref, seed_ref, pos256_ref,
                   rel_ref, knn_ref, relm_ref, pencm_ref):
    b = pl.program_id(0)
    i = pl.program_id(1)
    q = posT_ref[0]
    s = seed_ref[0]
    p256 = pos256_ref[0]

    qb = q.astype(jnp.bfloat16).astype(_F32)
    sb = s.astype(jnp.bfloat16).astype(_F32)
    nq = (q[:, 0:1] * q[:, 0:1] + q[:, 1:2] * q[:, 1:2]) + q[:, 2:3] * q[:, 2:3]
    ns = (s[0:1, :] * s[0:1, :] + s[1:2, :] * s[1:2, :]) + s[2:3, :] * s[2:3, :]
    dot = ((qb[:, 0:1] * sb[0:1, :] + qb[:, 1:2] * sb[1:2, :])
           + qb[:, 2:3] * sb[2:3, :])
    d = (nq - 2.0 * dot) + ns

    iota = jax.lax.broadcasted_iota(jnp.int32, (NB_A, M), 1).astype(_F32)
    knn_parts = []
    for _ in range(K):
        mn = jnp.min(d, axis=1, keepdims=True)
        idx = jnp.min(jnp.where(d == mn, iota, _F32(M)), axis=1, keepdims=True)
        oh = (iota == idx).astype(_F32)
        knn_k = jax.lax.dot_general(oh, p256, (((1,), (1,)), ((), ())),
                                    preferred_element_type=_F32, precision=jax.lax.Precision.HIGHEST)
        knn_parts.append(knn_k)
        d = jnp.where(iota == idx, jnp.inf, d)

    knn = jnp.concatenate(knn_parts, axis=0)
    qt = jnp.concatenate([q] * K, axis=0)
    rel = qt - knn
    rel_ref[...] = rel
    knn_ref[...] = knn

    sel, freqs, is_cos = _enc_consts(_F32)
    penc = _posenc(knn, sel, freqs, is_cos)

    rel_sum = jnp.sum(rel, axis=0, keepdims=True)
    rel_gram = jax.lax.dot_general(rel, rel, (((0,), (0,)), ((), ())),
                                   preferred_element_type=_F32, precision=jax.lax.Precision.HIGHEST)
    penc_sum = jnp.sum(penc, axis=0, keepdims=True)
    penc_gram = jax.lax.dot_general(penc, penc, (((0,), (0,)), ((), ())),
                                    preferred_element_type=_F32, precision=jax.lax.Precision.HIGHEST)

    relm = jnp.concatenate([rel_sum, rel_gram], axis=0)
    pencm = jnp.concatenate([penc_sum, penc_gram], axis=0)

    @pl.when((b == 0) & (i == 0))
    def _():
        relm_ref[...] = relm
        pencm_ref[...] = pencm

    @pl.when((b > 0) | (i > 0))
    def _():
        relm_ref[...] += relm
        pencm_ref[...] += pencm


def _pass_b_kernel(rel_ref, A1_ref, c1_ref, a1m_ref):
    i = pl.program_id(0)
    rel = rel_ref[...]
    a1 = jnp.maximum(
        jax.lax.dot_general(rel, A1_ref[...], (((1,), (0,)), ((), ())),
                            preferred_element_type=_F32, precision=jax.lax.Precision.HIGHEST) + c1_ref[...], 0.0)
    a1_sum = jnp.sum(a1, axis=0, keepdims=True)
    a1_gram = jax.lax.dot_general(a1, a1, (((0,), (0,)), ((), ())),
                                  preferred_element_type=_F32, precision=jax.lax.Precision.HIGHEST)
    a1m = jnp.concatenate([a1_sum, a1_gram], axis=0)

    @pl.when(i == 0)
    def _():
        a1m_ref[...] = a1m

    @pl.when(i > 0)
    def _():
        a1m_ref[...] += a1m


def _pass_c_kernel(rel_ref, knn_ref,
                   A1_ref, c1_ref, A2_ref, b2_ref, A3_ref, c3_ref,
                   A4_ref, b4_ref, A5_ref, c5_ref, A6_ref, b6_ref,
                   out_ref):
    rel = rel_ref[...]
    knn = knn_ref[...]

    mm = lambda x, w: jax.lax.dot_general(x, w, (((1,), (0,)), ((), ())),
                                          preferred_element_type=_F32, precision=jax.lax.Precision.HIGHEST)

    a1 = jnp.maximum(mm(rel, A1_ref[...]) + c1_ref[...], 0.0)
    pe = mm(a1, A2_ref[...]) + b2_ref[...]
    w3 = jnp.maximum(mm(pe, A3_ref[...]) + c3_ref[...], 0.0)
    w4 = mm(w3, A4_ref[...]) + b4_ref[...]

    sel, freqs, is_cos = _enc_consts(_F32)
    penc = _posenc(knn, sel, freqs, is_cos)
    f5 = jnp.maximum(mm(penc, A5_ref[...]) + c5_ref[...], 0.0)
    f6 = mm(f5, A6_ref[...]) + b6_ref[...] + pe

    wk = [w4[k * NB_C:(k + 1) * NB_C, :] for k in range(K)]
    mx = wk[0]
    for k in range(1, K):
        mx = jnp.maximum(mx, wk[k])
    ek = [jnp.exp(wk[k] - mx) for k in range(K)]
    den = ek[0]
    for k in range(1, K):
        den = den + ek[k]
    acc = jnp.zeros((NB_C, 128), _F32)
    for k in range(K):
        acc = acc + ek[k] * f6[k * NB_C:(k + 1) * NB_C, :]
    out_ref[...] = acc / den


@functools.partial(jax.jit, static_argnums=())
def kernel(pos, seed, W1, b1, g1, be1, W2, b2, W3, b3, g3, be3, W4, b4,
           W5, b5, g5, be5, W6, b6):
    P = B * N * K
    nbn = N // NB_A
    posT = pos.transpose(0, 2, 1)
    pos256 = pos[:, :, :M]

    rel, knn, relm, pencm = pl.pallas_call(
        _pass_a_kernel,
        grid=(B, nbn),
        in_specs=[
            pl.BlockSpec((1, NB_A, 3), lambda b, i: (b, i, 0)),
            pl.BlockSpec((1, 3, M), lambda b, i: (b, 0, 0)),
            pl.BlockSpec((1, 3, M), lambda b, i: (b, 0, 0)),
        ],
        out_specs=[
            pl.BlockSpec((K * NB_A, 3), lambda b, i: (b * (N // NB_A) + i, 0)),
            pl.BlockSpec((K * NB_A, 3), lambda b, i: (b * (N // NB_A) + i, 0)),
            pl.BlockSpec((4, 3), lambda b, i: (0, 0)),
            pl.BlockSpec((61, CENC), lambda b, i: (0, 0)),
        ],
        out_shape=[
            jax.ShapeDtypeStruct((P, 3), _F32),
            jax.ShapeDtypeStruct((P, 3), _F32),
            jax.ShapeDtypeStruct((4, 3), _F32),
            jax.ShapeDtypeStruct((61, CENC), _F32),
        ],
    )(posT, seed, pos256)

    Pf = _F32(P)
    mu_rel = relm[0:1, :] / Pf
    C_rel = relm[1:4, :] / Pf - mu_rel.T @ mu_rel
    mean1 = W1 @ mu_rel[0] + b1
    var1 = jnp.sum((W1 @ C_rel) * W1, axis=1)
    s1 = g1 / jnp.sqrt(var1 + EPS)
    A1 = W1.T * s1[None, :]
    c1 = ((b1 - mean1) * s1 + be1)[None, :]

    mu_penc = pencm[0:1, :] / Pf
    C_penc = pencm[1:61, :] / Pf - mu_penc.T @ mu_penc
    mean5 = W5 @ mu_penc[0] + b5
    var5 = jnp.sum((W5 @ C_penc) * W5, axis=1)
    s5 = g5 / jnp.sqrt(var5 + EPS)
    A5 = W5.T * s5[None, :]
    c5 = ((b5 - mean5) * s5 + be5)[None, :]

    a1m = pl.pallas_call(
        _pass_b_kernel,
        grid=(P // PB_B,),
        in_specs=[
            pl.BlockSpec((PB_B, 3), lambda i: (i, 0)),
            pl.BlockSpec((3, 64), lambda i: (0, 0)),
            pl.BlockSpec((1, 64), lambda i: (0, 0)),
        ],
        out_specs=pl.BlockSpec((65, 64), lambda i: (0, 0)),
        out_shape=jax.ShapeDtypeStruct((65, 64), _F32),
    )(rel, A1, c1)

    mu_a1 = a1m[0:1, :] / Pf
    C_a1 = a1m[1:65, :] / Pf - mu_a1.T @ mu_a1
    Mw = W3 @ W2
    mean3 = Mw @ mu_a1[0] + W3 @ b2 + b3
    var3 = jnp.sum((Mw @ C_a1) * Mw, axis=1)
    s3 = g3 / jnp.sqrt(var3 + EPS)
    A3 = W3.T * s3[None, :]
    c3 = ((b3 - mean3) * s3 + be3)[None, :]

    A2 = W2.T
    A4 = W4.T
    A6 = W6.T

    out = pl.pallas_call(
        _pass_c_kernel,
        grid=(P // PB_C,),
        in_specs=[
            pl.BlockSpec((PB_C, 3), lambda i: (i, 0)),
            pl.BlockSpec((PB_C, 3), lambda i: (i, 0)),
            pl.BlockSpec((3, 64), lambda i: (0, 0)),
            pl.BlockSpec((1, 64), lambda i: (0, 0)),
            pl.BlockSpec((64, 128), lambda i: (0, 0)),
            pl.BlockSpec((1, 128), lambda i: (0, 0)),
            pl.BlockSpec((128, 512), lambda i: (0, 0)),
            pl.BlockSpec((1, 512), lambda i: (0, 0)),
            pl.BlockSpec((512, 128), lambda i: (0, 0)),
            pl.BlockSpec((1, 128), lambda i: (0, 0)),
            pl.BlockSpec((CENC, 128), lambda i: (0, 0)),
            pl.BlockSpec((1, 128), lambda i: (0, 0)),
            pl.BlockSpec((128, 128), lambda i: (0, 0)),
            pl.BlockSpec((1, 128), lambda i: (0, 0)),
        ],
        out_specs=pl.BlockSpec((NB_C, 128), lambda i: (i, 0)),
        out_shape=jax.ShapeDtypeStruct((B * N, 128), _F32),
    )(rel, knn, A1, c1, A2, b2[None, :], A3, c3, A4, b4[None, :],
      A5, c5, A6, b6[None, :])

    return out.reshape(B, N, 128).transpose(0, 2, 1)


# speedup vs baseline: 5.0085x; 3.0116x over previous
import functools

import jax
import jax.numpy as jnp
from jax.experimental import pallas as pl

B = 4
N = 2048
M = 256
K = 8
CENC = 60
L = 10
EPS = 1e-5

NB_A = 256
PB_B = 4096
NB_C = 256
PB_C = NB_C * K

_F32 = jnp.float32
_BF16 = jnp.bfloat16


def _dot(x, w):
    return jax.lax.dot_general(x, w, (((1,), (0,)), ((), ())),
                               preferred_element_type=_F32)


def _gather_dot(oh, w):
    hi = w.astype(_BF16).astype(_F32)
    r = w - hi
    mid = r.astype(_BF16).astype(_F32)
    lo = r - mid
    dg = lambda a, b: jax.lax.dot_general(a, b, (((1,), (1,)), ((), ())),
                                          preferred_element_type=_F32)
    return dg(oh, hi) + dg(oh, mid) + dg(oh, lo)


def _posenc(knn, freqs, phase):
    P_ = knn.shape[0]
    xb = jnp.concatenate(
        [jnp.broadcast_to(knn[:, c:c + 1], (P_, 20)) for c in range(3)], axis=1)
    xf = xb * freqs
    return jnp.sin(xf + phase)


def _enc_consts(dtype):
    j = jax.lax.broadcasted_iota(jnp.int32, (1, CENC), 1)
    freqs = jnp.round(jnp.exp(0.6931471805599453 * ((j % 20) % L).astype(dtype)))
    phase = jnp.where((j % 20) >= L, dtype(1.5707963267948966), dtype(0.0))
    return freqs, phase


def _pass_a_kernel(posT_ref, seed_ref, pos256_ref,
                   rel_ref, penc_ref, relm_ref, pencm_ref):
    b = pl.program_id(0)
    i = pl.program_id(1)
    q = posT_ref[0]
    s = seed_ref[0]
    p256 = pos256_ref[0]

    qb = q.astype(jnp.bfloat16).astype(_F32)
    sb = s.astype(jnp.bfloat16).astype(_F32)
    nq = (q[:, 0:1] * q[:, 0:1] + q[:, 1:2] * q[:, 1:2]) + q[:, 2:3] * q[:, 2:3]
    ns = (s[0:1, :] * s[0:1, :] + s[1:2, :] * s[1:2, :]) + s[2:3, :] * s[2:3, :]
    dot = ((qb[:, 0:1] * sb[0:1, :] + qb[:, 1:2] * sb[1:2, :])
           + qb[:, 2:3] * sb[2:3, :])
    d = (nq - 2.0 * dot) + ns

    iota = jax.lax.broadcasted_iota(jnp.int32, (NB_A, M), 1).astype(_F32)
    knn_parts = []
    for _ in range(K):
        mn = jnp.min(d, axis=1, keepdims=True)
        idx = jnp.min(jnp.where(d == mn, iota, _F32(M)), axis=1, keepdims=True)
        oh = (iota == idx).astype(_F32)
        knn_k = _gather_dot(oh, p256)
        knn_parts.append(knn_k)
        d = jnp.where(iota == idx, jnp.inf, d)

    knn = jnp.concatenate(knn_parts, axis=0)
    qt = jnp.concatenate([q] * K, axis=0)
    rel = qt - knn
    rel_ref[...] = rel

    freqs, phase = _enc_consts(_F32)
    penc = _posenc(knn, freqs, phase)
    penc_ref[...] = penc

    rel_sum = jnp.sum(rel, axis=0, keepdims=True)
    rel_gram = jax.lax.dot_general(rel, rel, (((0,), (0,)), ((), ())),
                                   preferred_element_type=_F32)
    penc_sum = jnp.sum(penc, axis=0, keepdims=True)
    penc_gram = jax.lax.dot_general(penc, penc, (((0,), (0,)), ((), ())),
                                    preferred_element_type=_F32)

    relm = jnp.concatenate([rel_sum, rel_gram], axis=0)
    pencm = jnp.concatenate([penc_sum, penc_gram], axis=0)

    @pl.when((b == 0) & (i == 0))
    def _():
        relm_ref[...] = relm
        pencm_ref[...] = pencm

    @pl.when((b > 0) | (i > 0))
    def _():
        relm_ref[...] += relm
        pencm_ref[...] += pencm


def _pass_b_kernel(rel_ref, A1_ref, c1_ref, a1m_ref):
    i = pl.program_id(0)
    rel = rel_ref[...]
    a1 = jnp.maximum(_dot(rel, A1_ref[...]) + c1_ref[...], 0.0)
    a1_sum = jnp.sum(a1, axis=0, keepdims=True)
    a1_gram = jax.lax.dot_general(a1, a1, (((0,), (0,)), ((), ())),
                                  preferred_element_type=_F32)
    a1m = jnp.concatenate([a1_sum, a1_gram], axis=0)

    @pl.when(i == 0)
    def _():
        a1m_ref[...] = a1m

    @pl.when(i > 0)
    def _():
        a1m_ref[...] += a1m


def _pass_c_kernel(rel_ref, penc_ref,
                   A1_ref, c1_ref, A2_ref, b2_ref, A3_ref, c3_ref,
                   A4_ref, b4_ref, A5_ref, c5_ref, A6_ref, b6_ref,
                   out_ref):
    rel = rel_ref[...]
    penc = penc_ref[...]

    mm = _dot

    a1 = jnp.maximum(mm(rel, A1_ref[...]) + c1_ref[...], 0.0)
    pe = mm(a1, A2_ref[...]) + b2_ref[...]
    w3 = jnp.maximum(mm(pe, A3_ref[...]) + c3_ref[...], 0.0)
    w4 = mm(w3, A4_ref[...]) + b4_ref[...]

    f5 = jnp.maximum(mm(penc, A5_ref[...]) + c5_ref[...], 0.0)
    f6 = mm(f5, A6_ref[...]) + b6_ref[...] + pe

    wk = [w4[k * NB_C:(k + 1) * NB_C, :] for k in range(K)]
    mx = wk[0]
    for k in range(1, K):
        mx = jnp.maximum(mx, wk[k])
    ek = [jnp.exp(wk[k] - mx) for k in range(K)]
    den = ek[0]
    for k in range(1, K):
        den = den + ek[k]
    acc = jnp.zeros((NB_C, 128), _F32)
    for k in range(K):
        acc = acc + ek[k] * f6[k * NB_C:(k + 1) * NB_C, :]
    out_ref[...] = acc / den


@functools.partial(jax.jit, static_argnums=())
def kernel(pos, seed, W1, b1, g1, be1, W2, b2, W3, b3, g3, be3, W4, b4,
           W5, b5, g5, be5, W6, b6):
    P = B * N * K
    nbn = N // NB_A
    posT = pos.transpose(0, 2, 1)
    pos256 = pos[:, :, :M]

    rel, penc, relm, pencm = pl.pallas_call(
        _pass_a_kernel,
        grid=(B, nbn),
        in_specs=[
            pl.BlockSpec((1, NB_A, 3), lambda b, i: (b, i, 0)),
            pl.BlockSpec((1, 3, M), lambda b, i: (b, 0, 0)),
            pl.BlockSpec((1, 3, M), lambda b, i: (b, 0, 0)),
        ],
        out_specs=[
            pl.BlockSpec((K * NB_A, 3), lambda b, i: (b * (N // NB_A) + i, 0)),
            pl.BlockSpec((K * NB_A, CENC), lambda b, i: (b * (N // NB_A) + i, 0)),
            pl.BlockSpec((4, 3), lambda b, i: (0, 0)),
            pl.BlockSpec((61, CENC), lambda b, i: (0, 0)),
        ],
        out_shape=[
            jax.ShapeDtypeStruct((P, 3), _F32),
            jax.ShapeDtypeStruct((P, CENC), _F32),
            jax.ShapeDtypeStruct((4, 3), _F32),
            jax.ShapeDtypeStruct((61, CENC), _F32),
        ],
    )(posT, seed, pos256)

    Pf = _F32(P)
    mu_rel = relm[0:1, :] / Pf
    C_rel = relm[1:4, :] / Pf - mu_rel.T @ mu_rel
    mean1 = W1 @ mu_rel[0] + b1
    var1 = jnp.sum((W1 @ C_rel) * W1, axis=1)
    s1 = g1 / jnp.sqrt(var1 + EPS)
    A1 = W1.T * s1[None, :]
    c1 = ((b1 - mean1) * s1 + be1)[None, :]

    mu_penc = pencm[0:1, :] / Pf
    C_penc = pencm[1:61, :] / Pf - mu_penc.T @ mu_penc
    mean5 = W5 @ mu_penc[0] + b5
    var5 = jnp.sum((W5 @ C_penc) * W5, axis=1)
    s5 = g5 / jnp.sqrt(var5 + EPS)
    A5 = W5.T * s5[None, :]
    c5 = ((b5 - mean5) * s5 + be5)[None, :]

    a1m = pl.pallas_call(
        _pass_b_kernel,
        grid=(P // PB_B,),
        in_specs=[
            pl.BlockSpec((PB_B, 3), lambda i: (i, 0)),
            pl.BlockSpec((3, 64), lambda i: (0, 0)),
            pl.BlockSpec((1, 64), lambda i: (0, 0)),
        ],
        out_specs=pl.BlockSpec((65, 64), lambda i: (0, 0)),
        out_shape=jax.ShapeDtypeStruct((65, 64), _F32),
    )(rel, A1, c1)

    mu_a1 = a1m[0:1, :] / Pf
    C_a1 = a1m[1:65, :] / Pf - mu_a1.T @ mu_a1
    Mw = W3 @ W2
    mean3 = Mw @ mu_a1[0] + W3 @ b2 + b3
    var3 = jnp.sum((Mw @ C_a1) * Mw, axis=1)
    s3 = g3 / jnp.sqrt(var3 + EPS)
    A3 = W3.T * s3[None, :]
    c3 = ((b3 - mean3) * s3 + be3)[None, :]

    A2 = W2.T
    A4 = W4.T
    A6 = W6.T

    out = pl.pallas_call(
        _pass_c_kernel,
        grid=(P // PB_C,),
        in_specs=[
            pl.BlockSpec((PB_C, 3), lambda i: (i, 0)),
            pl.BlockSpec((PB_C, CENC), lambda i: (i, 0)),
            pl.BlockSpec((3, 64), lambda i: (0, 0)),
            pl.BlockSpec((1, 64), lambda i: (0, 0)),
            pl.BlockSpec((64, 128), lambda i: (0, 0)),
            pl.BlockSpec((1, 128), lambda i: (0, 0)),
            pl.BlockSpec((128, 512), lambda i: (0, 0)),
            pl.BlockSpec((1, 512), lambda i: (0, 0)),
            pl.BlockSpec((512, 128), lambda i: (0, 0)),
            pl.BlockSpec((1, 128), lambda i: (0, 0)),
            pl.BlockSpec((CENC, 128), lambda i: (0, 0)),
            pl.BlockSpec((1, 128), lambda i: (0, 0)),
            pl.BlockSpec((128, 128), lambda i: (0, 0)),
            pl.BlockSpec((1, 128), lambda i: (0, 0)),
        ],
        out_specs=pl.BlockSpec((NB_C, 128), lambda i: (i, 0)),
        out_shape=jax.ShapeDtypeStruct((B * N, 128), _F32),
    )(rel, penc, A1, c1, A2, b2[None, :], A3, c3, A4, b4[None, :],
      A5, c5, A6, b6[None, :])

    return out.reshape(B, N, 128).transpose(0, 2, 1)


# speedup vs baseline: 6.9511x; 1.3879x over previous
import functools

import jax
import jax.numpy as jnp
from jax.experimental import pallas as pl

B = 4
N = 2048
M = 256
K = 8
CENC = 60
L = 10
EPS = 1e-5

NB_A = 256
PB_B = 4096
NB_C = 256
PB_C = NB_C * K

_F32 = jnp.float32
_BF16 = jnp.bfloat16


def _dot(x, w):
    return jax.lax.dot_general(x, w, (((1,), (0,)), ((), ())),
                               preferred_element_type=_F32)


def _gather_dot(oh, w):
    hi = w.astype(_BF16).astype(_F32)
    r = w - hi
    mid = r.astype(_BF16).astype(_F32)
    lo = r - mid
    dg = lambda a, b: jax.lax.dot_general(a, b, (((1,), (1,)), ((), ())),
                                          preferred_element_type=_F32)
    return dg(oh, hi) + dg(oh, mid) + dg(oh, lo)


_INV2PI = 0.15915493667125702
_C1 = 6.28125
_C2 = 0.0019353071693331003
_C3 = 1.0253131677018246e-11
_SIN_COEF = (0.9999995827674866, -0.1666654646396637, 0.008332370780408382,
             -0.00019807845819741488, 2.69886936621333e-06,
             -2.03291836697872e-08)


def _fast_sin(x):
    n = jnp.floor(x * _INV2PI + 0.5)
    r = ((x - n * _C1) - n * _C2) - n * _C3
    u = r * r
    p = _F32(_SIN_COEF[5])
    for c in (_SIN_COEF[4], _SIN_COEF[3], _SIN_COEF[2], _SIN_COEF[1],
              _SIN_COEF[0]):
        p = p * u + c
    return r * p


def _posenc(knn, freqs, phase):
    P_ = knn.shape[0]
    xb = jnp.concatenate(
        [jnp.broadcast_to(knn[:, c:c + 1], (P_, 20)) for c in range(3)], axis=1)
    xf = xb * freqs
    return _fast_sin(xf + phase)


def _enc_consts(dtype):
    j = jax.lax.broadcasted_iota(jnp.int32, (1, CENC), 1)
    freqs = jnp.round(jnp.exp(0.6931471805599453 * ((j % 20) % L).astype(dtype)))
    phase = jnp.where((j % 20) >= L, dtype(1.5707963267948966), dtype(0.0))
    return freqs, phase


def _pass_a_kernel(posT_ref, seed_ref, pos256_ref,
                   rel_ref, penc_ref, relm_ref, pencm_ref):
    b = pl.program_id(0)
    i = pl.program_id(1)
    q = posT_ref[0]
    s = seed_ref[0]
    p256 = pos256_ref[0]

    qb = q.astype(jnp.bfloat16).astype(_F32)
    sb = s.astype(jnp.bfloat16).astype(_F32)
    nq = (q[:, 0:1] * q[:, 0:1] + q[:, 1:2] * q[:, 1:2]) + q[:, 2:3] * q[:, 2:3]
    ns = (s[0:1, :] * s[0:1, :] + s[1:2, :] * s[1:2, :]) + s[2:3, :] * s[2:3, :]
    dot = ((qb[:, 0:1] * sb[0:1, :] + qb[:, 1:2] * sb[1:2, :])
           + qb[:, 2:3] * sb[2:3, :])
    d = (nq - 2.0 * dot) + ns

    iota = jax.lax.broadcasted_iota(jnp.int32, (NB_A, M), 1).astype(_F32)
    knn_parts = []
    for _ in range(K):
        mn = jnp.min(d, axis=1, keepdims=True)
        idx = jnp.min(jnp.where(d == mn, iota, _F32(M)), axis=1, keepdims=True)
        oh = (iota == idx).astype(_F32)
        knn_k = _gather_dot(oh, p256)
        knn_parts.append(knn_k)
        d = jnp.where(iota == idx, jnp.inf, d)

    knn = jnp.concatenate(knn_parts, axis=0)
    qt = jnp.concatenate([q] * K, axis=0)
    rel = qt - knn
    rel_ref[...] = rel

    freqs, phase = _enc_consts(_F32)
    penc = _posenc(knn, freqs, phase)
    penc_ref[...] = penc

    rel_sum = jnp.sum(rel, axis=0, keepdims=True)
    rel_gram = jax.lax.dot_general(rel, rel, (((0,), (0,)), ((), ())),
                                   preferred_element_type=_F32)
    penc_sum = jnp.sum(penc, axis=0, keepdims=True)
    penc_gram = jax.lax.dot_general(penc, penc, (((0,), (0,)), ((), ())),
                                    preferred_element_type=_F32)

    relm = jnp.concatenate([rel_sum, rel_gram], axis=0)
    pencm = jnp.concatenate([penc_sum, penc_gram], axis=0)

    @pl.when((b == 0) & (i == 0))
    def _():
        relm_ref[...] = relm
        pencm_ref[...] = pencm

    @pl.when((b > 0) | (i > 0))
    def _():
        relm_ref[...] += relm
        pencm_ref[...] += pencm


def _pass_b_kernel(rel_ref, A1_ref, c1_ref, a1m_ref):
    i = pl.program_id(0)
    rel = rel_ref[...]
    a1 = jnp.maximum(_dot(rel, A1_ref[...]) + c1_ref[...], 0.0)
    a1_sum = jnp.sum(a1, axis=0, keepdims=True)
    a1_gram = jax.lax.dot_general(a1, a1, (((0,), (0,)), ((), ())),
                                  preferred_element_type=_F32)
    a1m = jnp.concatenate([a1_sum, a1_gram], axis=0)

    @pl.when(i == 0)
    def _():
        a1m_ref[...] = a1m

    @pl.when(i > 0)
    def _():
        a1m_ref[...] += a1m


def _pass_c_kernel(rel_ref, penc_ref,
                   A1_ref, c1_ref, A2_ref, b2_ref, A3_ref, c3_ref,
                   A4_ref, b4_ref, A5_ref, c5_ref, A6_ref, b6_ref,
                   out_ref):
    rel = rel_ref[...]
    penc = penc_ref[...]

    mm = _dot

    a1 = jnp.maximum(mm(rel, A1_ref[...]) + c1_ref[...], 0.0)
    pe = mm(a1, A2_ref[...]) + b2_ref[...]
    w3 = jnp.maximum(mm(pe, A3_ref[...]) + c3_ref[...], 0.0)
    w4 = mm(w3, A4_ref[...]) + b4_ref[...]

    f5 = jnp.maximum(mm(penc, A5_ref[...]) + c5_ref[...], 0.0)
    f6 = mm(f5, A6_ref[...]) + b6_ref[...] + pe

    wk = [w4[k * NB_C:(k + 1) * NB_C, :] for k in range(K)]
    mx = wk[0]
    for k in range(1, K):
        mx = jnp.maximum(mx, wk[k])
    ek = [jnp.exp(wk[k] - mx) for k in range(K)]
    den = ek[0]
    for k in range(1, K):
        den = den + ek[k]
    acc = jnp.zeros((NB_C, 128), _F32)
    for k in range(K):
        acc = acc + ek[k] * f6[k * NB_C:(k + 1) * NB_C, :]
    out_ref[...] = acc / den


@functools.partial(jax.jit, static_argnums=())
def kernel(pos, seed, W1, b1, g1, be1, W2, b2, W3, b3, g3, be3, W4, b4,
           W5, b5, g5, be5, W6, b6):
    P = B * N * K
    nbn = N // NB_A
    posT = pos.transpose(0, 2, 1)
    pos256 = pos[:, :, :M]

    rel, penc, relm, pencm = pl.pallas_call(
        _pass_a_kernel,
        grid=(B, nbn),
        in_specs=[
            pl.BlockSpec((1, NB_A, 3), lambda b, i: (b, i, 0)),
            pl.BlockSpec((1, 3, M), lambda b, i: (b, 0, 0)),
            pl.BlockSpec((1, 3, M), lambda b, i: (b, 0, 0)),
        ],
        out_specs=[
            pl.BlockSpec((K * NB_A, 3), lambda b, i: (b * (N // NB_A) + i, 0)),
            pl.BlockSpec((K * NB_A, CENC), lambda b, i: (b * (N // NB_A) + i, 0)),
            pl.BlockSpec((4, 3), lambda b, i: (0, 0)),
            pl.BlockSpec((61, CENC), lambda b, i: (0, 0)),
        ],
        out_shape=[
            jax.ShapeDtypeStruct((P, 3), _F32),
            jax.ShapeDtypeStruct((P, CENC), _F32),
            jax.ShapeDtypeStruct((4, 3), _F32),
            jax.ShapeDtypeStruct((61, CENC), _F32),
        ],
    )(posT, seed, pos256)

    Pf = _F32(P)
    mu_rel = relm[0:1, :] / Pf
    C_rel = relm[1:4, :] / Pf - mu_rel.T @ mu_rel
    mean1 = W1 @ mu_rel[0] + b1
    var1 = jnp.sum((W1 @ C_rel) * W1, axis=1)
    s1 = g1 / jnp.sqrt(var1 + EPS)
    A1 = W1.T * s1[None, :]
    c1 = ((b1 - mean1) * s1 + be1)[None, :]

    mu_penc = pencm[0:1, :] / Pf
    C_penc = pencm[1:61, :] / Pf - mu_penc.T @ mu_penc
    mean5 = W5 @ mu_penc[0] + b5
    var5 = jnp.sum((W5 @ C_penc) * W5, axis=1)
    s5 = g5 / jnp.sqrt(var5 + EPS)
    A5 = W5.T * s5[None, :]
    c5 = ((b5 - mean5) * s5 + be5)[None, :]

    a1m = pl.pallas_call(
        _pass_b_kernel,
        grid=(P // PB_B,),
        in_specs=[
            pl.BlockSpec((PB_B, 3), lambda i: (i, 0)),
            pl.BlockSpec((3, 64), lambda i: (0, 0)),
            pl.BlockSpec((1, 64), lambda i: (0, 0)),
        ],
        out_specs=pl.BlockSpec((65, 64), lambda i: (0, 0)),
        out_shape=jax.ShapeDtypeStruct((65, 64), _F32),
    )(rel, A1, c1)

    mu_a1 = a1m[0:1, :] / Pf
    C_a1 = a1m[1:65, :] / Pf - mu_a1.T @ mu_a1
    Mw = W3 @ W2
    mean3 = Mw @ mu_a1[0] + W3 @ b2 + b3
    var3 = jnp.sum((Mw @ C_a1) * Mw, axis=1)
    s3 = g3 / jnp.sqrt(var3 + EPS)
    A3 = W3.T * s3[None, :]
    c3 = ((b3 - mean3) * s3 + be3)[None, :]

    A2 = W2.T
    A4 = W4.T
    A6 = W6.T

    out = pl.pallas_call(
        _pass_c_kernel,
        grid=(P // PB_C,),
        in_specs=[
            pl.BlockSpec((PB_C, 3), lambda i: (i, 0)),
            pl.BlockSpec((PB_C, CENC), lambda i: (i, 0)),
            pl.BlockSpec((3, 64), lambda i: (0, 0)),
            pl.BlockSpec((1, 64), lambda i: (0, 0)),
            pl.BlockSpec((64, 128), lambda i: (0, 0)),
            pl.BlockSpec((1, 128), lambda i: (0, 0)),
            pl.BlockSpec((128, 512), lambda i: (0, 0)),
            pl.BlockSpec((1, 512), lambda i: (0, 0)),
            pl.BlockSpec((512, 128), lambda i: (0, 0)),
            pl.BlockSpec((1, 128), lambda i: (0, 0)),
            pl.BlockSpec((CENC, 128), lambda i: (0, 0)),
            pl.BlockSpec((1, 128), lambda i: (0, 0)),
            pl.BlockSpec((128, 128), lambda i: (0, 0)),
            pl.BlockSpec((1, 128), lambda i: (0, 0)),
        ],
        out_specs=pl.BlockSpec((NB_C, 128), lambda i: (i, 0)),
        out_shape=jax.ShapeDtypeStruct((B * N, 128), _F32),
    )(rel, penc, A1, c1, A2, b2[None, :], A3, c3, A4, b4[None, :],
      A5, c5, A6, b6[None, :])

    return out.reshape(B, N, 128).transpose(0, 2, 1)
